# Initial kernel scaffold; baseline (speedup 1.0000x reference)
import jax
import jax.numpy as jnp
from jax.experimental import pallas as pl


def kernel(x):
    raise NotImplementedError("write your pallas kernel here")


# speedup vs baseline: 3.5648x; 3.5648x over previous
import functools

import jax
import jax.numpy as jnp
from jax import lax
from jax.experimental import pallas as pl

B, S, C = 4, 8192, 1024
G = 16
GS = C // G
K = 256
S_T = 512
R = B * G


def _norms2_body(x_ref, o_ref):
    xb = x_ref[0]
    xsq = xb * xb
    rows = lax.broadcasted_iota(jnp.int32, (G, C), 0)
    cols = lax.broadcasted_iota(jnp.int32, (G, C), 1)
    e = (cols // GS == rows).astype(jnp.float32)
    o_ref[...] = lax.dot_general(
        e, xsq, (((1,), (1,)), ((), ())),
        preferred_element_type=jnp.float32,
        precision=lax.Precision.HIGHEST,
    )


def _select_body(n_ref, m_ref):
    bits = lax.bitcast_convert_type(n_ref[...], jnp.int32)
    kk = jnp.int32(K)

    def tbody(i, t):
        cand = t | (jnp.int32(1) << (30 - i))
        cnt = jnp.sum((bits >= cand).astype(jnp.int32), axis=1, keepdims=True)
        return jnp.where(cnt >= kk, cand, t)

    t = lax.fori_loop(0, 31, tbody, jnp.zeros((R, 1), jnp.int32))

    gt = bits > t
    eq = bits == t
    need = kk - jnp.sum(gt.astype(jnp.int32), axis=1, keepdims=True)
    idx = lax.broadcasted_iota(jnp.int32, (R, S), 1)

    def jbody(i, j):
        cand = j + (jnp.int32(1) << (12 - i))
        f = jnp.sum((eq & (idx < cand)).astype(jnp.int32), axis=1,
                    keepdims=True)
        return jnp.where(f < need, cand, j)

    j = lax.fori_loop(0, 13, jbody, jnp.zeros((R, 1), jnp.int32))

    m_ref[...] = (gt | (eq & (idx <= j))).astype(jnp.float32)


def _apply_body(x_ref, m_ref, o_ref):
    m = m_ref[...]
    rows = lax.broadcasted_iota(jnp.int32, (G, C), 0)
    cols = lax.broadcasted_iota(jnp.int32, (G, C), 1)
    e = (cols // GS == rows).astype(jnp.float32)
    mx = lax.dot_general(
        m, e, (((0,), (0,)), ((), ())),
        preferred_element_type=jnp.float32,
    )
    o_ref[0] = x_ref[0] * mx


@jax.jit
def kernel(x):
    norms2 = pl.pallas_call(
        _norms2_body,
        grid=(B, S // S_T),
        in_specs=[pl.BlockSpec((1, S_T, C), lambda b, s: (b, s, 0))],
        out_specs=pl.BlockSpec((G, S_T), lambda b, s: (b, s)),
        out_shape=jax.ShapeDtypeStruct((R, S), jnp.float32),
    )(x)

    mask = pl.pallas_call(
        _select_body,
        in_specs=[pl.BlockSpec((R, S), lambda: (0, 0))],
        out_specs=pl.BlockSpec((R, S), lambda: (0, 0)),
        out_shape=jax.ShapeDtypeStruct((R, S), jnp.float32),
    )(norms2)

    out = pl.pallas_call(
        _apply_body,
        grid=(B, S // S_T),
        in_specs=[
            pl.BlockSpec((1, S_T, C), lambda b, s: (b, s, 0)),
            pl.BlockSpec((G, S_T), lambda b, s: (b, s)),
        ],
        out_specs=pl.BlockSpec((1, S_T, C), lambda b, s: (b, s, 0)),
        out_shape=jax.ShapeDtypeStruct((B, S, C), jnp.float32),
    )(x, mask)
    return out
